# baseline (speedup 1.0000x reference)
import functools

import jax
import jax.numpy as jnp
from jax import lax
from jax.experimental import pallas as pl
from jax.experimental.pallas import tpu as pltpu
from jax.experimental.pallas import tpu_sc as plsc

_NCHUNK = 8


@functools.cache
def _make_gather(V, D, B):
    info = plsc.get_sparse_core_info()
    NC, NS = info.num_cores, info.num_subcores
    NW = NC * NS
    assert B % (8 * NW) == 0
    b_per_w = B // NW
    nch = _NCHUNK
    assert b_per_w % nch == 0
    w = b_per_w // nch
    mesh = plsc.VectorSubcoreMesh(core_axis_name="c", subcore_axis_name="s")

    @functools.partial(
        pl.kernel,
        mesh=mesh,
        out_type=jax.ShapeDtypeStruct((B, D), jnp.float32),
        scratch_types=[
            pltpu.VMEM((b_per_w,), jnp.int32),
            pltpu.VMEM((b_per_w, D), jnp.float32),
            pltpu.SemaphoreType.DMA((nch,)),
            pltpu.SemaphoreType.DMA,
        ],
    )
    def k(idx_hbm, table_hbm, out_hbm, idx_v, rows_v, in_sems, out_sem):
        wid = lax.axis_index("s") * NC + lax.axis_index("c")
        base = wid * b_per_w
        pltpu.sync_copy(idx_hbm.at[pl.ds(base, b_per_w)], idx_v)
        gathers = [
            pltpu.async_copy(
                table_hbm.at[idx_v.at[pl.ds(c * w, w)]],
                rows_v.at[pl.ds(c * w, w)],
                in_sems.at[c],
            )
            for c in range(nch)
        ]
        outs = []
        for c in range(nch):
            gathers[c].wait()
            outs.append(
                pltpu.async_copy(
                    rows_v.at[pl.ds(c * w, w)],
                    out_hbm.at[pl.ds(base + c * w, w)],
                    out_sem,
                )
            )
        for o in outs:
            o.wait()

    return k


def kernel(style_id, style_table):
    B = style_id.shape[0]
    V, D = style_table.shape
    g = _make_gather(V, D, B)
    return g(style_id.astype(jnp.int32), style_table)


# speedup vs baseline: 1.2097x; 1.2097x over previous
import functools

import jax
import jax.numpy as jnp
from jax import lax
from jax.experimental import pallas as pl
from jax.experimental.pallas import tpu as pltpu
from jax.experimental.pallas import tpu_sc as plsc

_NCHUNK = 8


@functools.cache
def _make_gather(Vp, D, B):
    info = plsc.get_sparse_core_info()
    NC, NS = info.num_cores, info.num_subcores
    NW = NC * NS
    assert B % (8 * NW) == 0 and Vp % NS == 0
    b_per_w = B // NW
    v_per_t = Vp // NS
    nch = _NCHUNK
    assert b_per_w % nch == 0
    w = b_per_w // nch
    mesh = plsc.VectorSubcoreMesh(core_axis_name="c", subcore_axis_name="s")

    @functools.partial(
        pl.kernel,
        mesh=mesh,
        out_type=jax.ShapeDtypeStruct((B, D), jnp.float32),
        scratch_types=[
            pltpu.VMEM((b_per_w,), jnp.int32),
            pltpu.VMEM((b_per_w, D), jnp.float32),
            pltpu.VMEM_SHARED((Vp, D), jnp.float32),
            pltpu.SemaphoreType.DMA((nch,)),
            pltpu.SemaphoreType.DMA,
        ],
    )
    def k(idx_hbm, table_hbm, out_hbm, idx_v, rows_v, table_sh, in_sems, out_sem):
        sid = lax.axis_index("s")
        wid = sid * NC + lax.axis_index("c")
        base = wid * b_per_w
        pltpu.sync_copy(
            table_hbm.at[pl.ds(sid * v_per_t, v_per_t)],
            table_sh.at[pl.ds(sid * v_per_t, v_per_t)],
        )
        pltpu.sync_copy(idx_hbm.at[pl.ds(base, b_per_w)], idx_v)
        plsc.subcore_barrier()
        gathers = [
            pltpu.async_copy(
                table_sh.at[idx_v.at[pl.ds(c * w, w)]],
                rows_v.at[pl.ds(c * w, w)],
                in_sems.at[c],
            )
            for c in range(nch)
        ]
        outs = []
        for c in range(nch):
            gathers[c].wait()
            outs.append(
                pltpu.async_copy(
                    rows_v.at[pl.ds(c * w, w)],
                    out_hbm.at[pl.ds(base + c * w, w)],
                    out_sem,
                )
            )
        for o in outs:
            o.wait()

    return k


def kernel(style_id, style_table):
    B = style_id.shape[0]
    V, D = style_table.shape
    Vp = -(-V // 128) * 128
    table = style_table
    if Vp != V:
        table = jnp.pad(style_table, ((0, Vp - V), (0, 0)))
    g = _make_gather(Vp, D, B)
    return g(style_id.astype(jnp.int32), table)


# speedup vs baseline: 1.2165x; 1.0056x over previous
import functools

import jax
import jax.numpy as jnp
from jax import lax
from jax.experimental import pallas as pl
from jax.experimental.pallas import tpu as pltpu
from jax.experimental.pallas import tpu_sc as plsc

_NCHUNK = 8
_VCHUNK = 64


@functools.cache
def _make_gather(V, D, B):
    info = plsc.get_sparse_core_info()
    NC, NS = info.num_cores, info.num_subcores
    NW = NC * NS
    assert B % (8 * NW) == 0
    b_per_w = B // NW
    nch = _NCHUNK
    assert b_per_w % nch == 0
    w = b_per_w // nch
    vch = _VCHUNK
    n_full = V // vch
    assert n_full <= NS
    v_rem = V - n_full * vch
    mesh = plsc.VectorSubcoreMesh(core_axis_name="c", subcore_axis_name="s")

    @functools.partial(
        pl.kernel,
        mesh=mesh,
        out_type=jax.ShapeDtypeStruct((B, D), jnp.float32),
        scratch_types=[
            pltpu.VMEM((b_per_w,), jnp.int32),
            pltpu.VMEM((b_per_w, D), jnp.float32),
            pltpu.VMEM_SHARED((V, D), jnp.float32),
            pltpu.SemaphoreType.DMA((nch,)),
            pltpu.SemaphoreType.DMA,
        ],
    )
    def k(idx_hbm, table_hbm, out_hbm, idx_v, rows_v, table_sh, in_sems, out_sem):
        sid = lax.axis_index("s")
        wid = sid * NC + lax.axis_index("c")
        base = wid * b_per_w
        @pl.when(sid < n_full)
        def _stage_full():
            pltpu.sync_copy(
                table_hbm.at[pl.ds(sid * vch, vch)],
                table_sh.at[pl.ds(sid * vch, vch)],
            )

        if v_rem:
            @pl.when(sid == n_full)
            def _stage_rem():
                pltpu.sync_copy(
                    table_hbm.at[pl.ds(n_full * vch, v_rem)],
                    table_sh.at[pl.ds(n_full * vch, v_rem)],
                )

        pltpu.sync_copy(idx_hbm.at[pl.ds(base, b_per_w)], idx_v)
        plsc.subcore_barrier()
        gathers = [
            pltpu.async_copy(
                table_sh.at[idx_v.at[pl.ds(c * w, w)]],
                rows_v.at[pl.ds(c * w, w)],
                in_sems.at[c],
            )
            for c in range(nch)
        ]
        outs = []
        for c in range(nch):
            gathers[c].wait()
            outs.append(
                pltpu.async_copy(
                    rows_v.at[pl.ds(c * w, w)],
                    out_hbm.at[pl.ds(base + c * w, w)],
                    out_sem,
                )
            )
        for o in outs:
            o.wait()

    return k


def kernel(style_id, style_table):
    B = style_id.shape[0]
    V, D = style_table.shape
    g = _make_gather(V, D, B)
    return g(style_id.astype(jnp.int32), style_table)
